# Initial kernel scaffold; baseline (speedup 1.0000x reference)
import jax
import jax.numpy as jnp
from jax.experimental import pallas as pl


def kernel(ind_2, prop, inter):
    raise NotImplementedError("write your pallas kernel here")


# speedup vs baseline: 31.0234x; 31.0234x over previous
import functools

import jax
import jax.numpy as jnp
from jax import lax
from jax.experimental import pallas as pl
from jax.experimental.pallas import tpu as pltpu
from jax.experimental.pallas import tpu_sc as plsc

_NC = 2
_NS = 16
_L = 16


def _scatter_body(n_rows, d, n_edges, w, ind2_hbm, inter_hbm, part_hbm,
                  acc, idx_buf, val_buf):
    g_cols = d // _L
    rows_half = n_rows // 2
    edges_half = n_edges // _NC
    n_windows = edges_half // w

    c = lax.axis_index("c")
    s = lax.axis_index("s")
    g = lax.rem(s, g_cols)
    rh = s // g_cols
    row0 = rh * rows_half
    col0 = g * _L

    zeros16 = jnp.zeros((_L,), jnp.float32)

    def zero_row(j, carry):
        acc[j, :] = zeros16
        return carry

    lax.fori_loop(0, rows_half, zero_row, 0)

    iota = lax.iota(jnp.int32, _L)

    def window(wi, carry):
        e0 = c * edges_half + wi * w
        pltpu.sync_copy(ind2_hbm.at[pl.ds(e0, w), 0, pl.ds(col0, _L)],
                        idx_buf)
        pltpu.sync_copy(inter_hbm.at[pl.ds(e0, w), pl.ds(col0, _L)],
                        val_buf)

        def edge(e, ecarry):
            iv = idx_buf[e, :]
            vv = val_buf[e, :]
            r = iv - row0
            mask = (r >= 0) & (r < rows_half)
            rc = jnp.where(mask, r, 0)
            plsc.addupdate_scatter(acc, [rc, iota], vv, mask=mask)
            return ecarry

        lax.fori_loop(0, w, edge, 0, unroll=8)
        return carry

    lax.fori_loop(0, n_windows, window, 0)

    pltpu.sync_copy(acc,
                    part_hbm.at[c, pl.ds(row0, rows_half), pl.ds(col0, _L)])


def _add_body(p_ref, o_ref):
    o_ref[...] = p_ref[0] + p_ref[1]


@jax.jit
def kernel(ind_2, prop, inter):
    n_rows, d = prop.shape
    n_edges = inter.shape[0]
    w = 1000

    mesh = plsc.VectorSubcoreMesh(core_axis_name="c", subcore_axis_name="s",
                                  num_cores=_NC, num_subcores=_NS)
    scatter = pl.kernel(
        functools.partial(_scatter_body, n_rows, d, n_edges, w),
        out_type=jax.ShapeDtypeStruct((_NC, n_rows, d), jnp.float32),
        mesh=mesh,
        scratch_types=[
            pltpu.VMEM((n_rows // 2, _L), jnp.float32),
            pltpu.VMEM((w, _L), jnp.int32),
            pltpu.VMEM((w, _L), jnp.float32),
        ],
        compiler_params=pltpu.CompilerParams(use_tc_tiling_on_sc=False,
                                             needs_layout_passes=False),
    )
    part = scatter(ind_2, inter)

    n_blocks = 10
    out = pl.pallas_call(
        _add_body,
        out_shape=jax.ShapeDtypeStruct((n_rows, d), jnp.float32),
        grid=(n_blocks,),
        in_specs=[pl.BlockSpec((_NC, n_rows // n_blocks, d),
                               lambda i: (0, i, 0))],
        out_specs=pl.BlockSpec((n_rows // n_blocks, d), lambda i: (i, 0)),
    )(part)
    return out


# speedup vs baseline: 121.6459x; 3.9211x over previous
import functools

import jax
import jax.numpy as jnp
from jax import lax
from jax.experimental import pallas as pl
from jax.experimental.pallas import tpu as pltpu
from jax.experimental.pallas import tpu_sc as plsc

_NC = 2
_NS = 16
_L = 16


def _scatter_body(n_rows, d, n_edges, w, ind2_hbm, inter_hbm, part_hbm,
                  acc, idx_buf, val_buf, si0, sv0, si1, sv1):
    g_cols = d // _L
    rows_half = n_rows // 2
    edges_half = n_edges // _NC
    n_windows = edges_half // w

    c = lax.axis_index("c")
    s = lax.axis_index("s")
    g = lax.rem(s, g_cols)
    rh = s // g_cols
    row0 = rh * rows_half
    col0 = g * _L
    half_u = jnp.uint32(rows_half)

    zeros16 = jnp.zeros((_L,), jnp.float32)

    def zero_row(j, carry):
        acc[j, :] = zeros16
        return carry

    lax.fori_loop(0, rows_half, zero_row, 0)

    iota = lax.iota(jnp.int32, _L)
    sis = (si0, si1)
    svs = (sv0, sv1)

    def idx_desc(wi, b):
        e0 = c * edges_half + wi * w
        return pltpu.make_async_copy(
            ind2_hbm.at[pl.ds(e0, w), 0, pl.ds(col0, _L)],
            idx_buf.at[b], sis[b])

    def val_desc(wi, b):
        e0 = c * edges_half + wi * w
        return pltpu.make_async_copy(
            inter_hbm.at[pl.ds(e0, w), pl.ds(col0, _L)],
            val_buf.at[b], svs[b])

    def start(wi, b):
        idx_desc(wi, b).start()
        val_desc(wi, b).start()

    start(0, 0)
    start(1, 1)

    def outer(o, carry):
        for b in range(2):
            wi = 2 * o + b
            idx_desc(wi, b).wait()
            val_desc(wi, b).wait()

            @plsc.parallel_loop(0, w, unroll=8)
            def edge(e):
                iv = idx_buf[b, e, :]
                vv = val_buf[b, e, :]
                r = iv - row0
                mask = plsc.bitcast(r, jnp.uint32) < half_u
                plsc.addupdate_scatter(acc, [r, iota], vv, mask=mask)

            @pl.when(wi + 2 < n_windows)
            def _():
                start(wi + 2, b)
        return carry

    lax.fori_loop(0, n_windows // 2, outer, 0)

    pltpu.sync_copy(acc,
                    part_hbm.at[c, pl.ds(row0, rows_half), pl.ds(col0, _L)])


def _add_body(p_ref, o_ref):
    o_ref[...] = p_ref[0] + p_ref[1]


@jax.jit
def kernel(ind_2, prop, inter):
    n_rows, d = prop.shape
    n_edges = inter.shape[0]
    w = 500

    mesh = plsc.VectorSubcoreMesh(core_axis_name="c", subcore_axis_name="s",
                                  num_cores=_NC, num_subcores=_NS)
    scatter = pl.kernel(
        functools.partial(_scatter_body, n_rows, d, n_edges, w),
        out_type=jax.ShapeDtypeStruct((_NC, n_rows, d), jnp.float32),
        mesh=mesh,
        scratch_types=[
            pltpu.VMEM((n_rows // 2, _L), jnp.float32),
            pltpu.VMEM((2, w, _L), jnp.int32),
            pltpu.VMEM((2, w, _L), jnp.float32),
            pltpu.SemaphoreType.DMA,
            pltpu.SemaphoreType.DMA,
            pltpu.SemaphoreType.DMA,
            pltpu.SemaphoreType.DMA,
        ],
        compiler_params=pltpu.CompilerParams(use_tc_tiling_on_sc=False,
                                             needs_layout_passes=False),
    )
    part = scatter(ind_2, inter)

    n_blocks = 10
    out = pl.pallas_call(
        _add_body,
        out_shape=jax.ShapeDtypeStruct((n_rows, d), jnp.float32),
        grid=(n_blocks,),
        in_specs=[pl.BlockSpec((_NC, n_rows // n_blocks, d),
                               lambda i: (0, i, 0))],
        out_specs=pl.BlockSpec((n_rows // n_blocks, d), lambda i: (i, 0)),
    )(part)
    return out


# speedup vs baseline: 124.6684x; 1.0248x over previous
import functools

import jax
import jax.numpy as jnp
from jax import lax
from jax.experimental import pallas as pl
from jax.experimental.pallas import tpu as pltpu
from jax.experimental.pallas import tpu_sc as plsc

_NC = 2
_NS = 16
_L = 16


def _scatter_body(n_rows, d, n_edges, w, ind2_hbm, inter_hbm, part_hbm,
                  acc, idx_buf, val_buf, si0, sv0, si1, sv1):
    g_cols = d // _L
    rows_half = n_rows // 2
    edges_half = n_edges // _NC
    n_windows = edges_half // w

    c = lax.axis_index("c")
    s = lax.axis_index("s")
    g = lax.rem(s, g_cols)
    rh = s // g_cols
    row0 = rh * rows_half
    col0 = g * _L
    half_u = jnp.uint32(rows_half)

    zeros16 = jnp.zeros((_L,), jnp.float32)

    def zero_row(j, carry):
        acc[j, :] = zeros16
        return carry

    lax.fori_loop(0, rows_half, zero_row, 0)

    iota = lax.iota(jnp.int32, _L)
    sis = (si0, si1)
    svs = (sv0, sv1)

    def idx_desc(wi, b):
        e0 = c * edges_half + wi * w
        return pltpu.make_async_copy(
            ind2_hbm.at[pl.ds(e0, w), 0, pl.ds(col0, _L)],
            idx_buf.at[b], sis[b])

    def val_desc(wi, b):
        e0 = c * edges_half + wi * w
        return pltpu.make_async_copy(
            inter_hbm.at[pl.ds(e0, w), pl.ds(col0, _L)],
            val_buf.at[b], svs[b])

    def start(wi, b):
        idx_desc(wi, b).start()
        val_desc(wi, b).start()

    start(0, 0)
    start(1, 1)

    def outer(o, carry):
        for b in range(2):
            wi = 2 * o + b
            idx_desc(wi, b).wait()
            val_desc(wi, b).wait()

            @plsc.parallel_loop(0, w, unroll=8)
            def edge(e):
                iv = idx_buf[b, e, :]
                vv = val_buf[b, e, :]
                r = iv - row0
                mask = plsc.bitcast(r, jnp.uint32) < half_u
                plsc.addupdate_scatter(acc, [r, iota], vv, mask=mask)

            @pl.when(wi + 2 < n_windows)
            def _():
                start(wi + 2, b)
        return carry

    lax.fori_loop(0, n_windows // 2, outer, 0)

    pltpu.sync_copy(acc,
                    part_hbm.at[c, pl.ds(row0, rows_half), pl.ds(col0, _L)])


def _add_body(p_ref, o_ref):
    o_ref[...] = p_ref[0] + p_ref[1]


@jax.jit
def kernel(ind_2, prop, inter):
    n_rows, d = prop.shape
    n_edges = inter.shape[0]
    w = 625

    mesh = plsc.VectorSubcoreMesh(core_axis_name="c", subcore_axis_name="s",
                                  num_cores=_NC, num_subcores=_NS)
    scatter = pl.kernel(
        functools.partial(_scatter_body, n_rows, d, n_edges, w),
        out_type=jax.ShapeDtypeStruct((_NC, n_rows, d), jnp.float32),
        mesh=mesh,
        scratch_types=[
            pltpu.VMEM((n_rows // 2, _L), jnp.float32),
            pltpu.VMEM((2, w, _L), jnp.int32),
            pltpu.VMEM((2, w, _L), jnp.float32),
            pltpu.SemaphoreType.DMA,
            pltpu.SemaphoreType.DMA,
            pltpu.SemaphoreType.DMA,
            pltpu.SemaphoreType.DMA,
        ],
        compiler_params=pltpu.CompilerParams(use_tc_tiling_on_sc=False,
                                             needs_layout_passes=False),
    )
    part = scatter(ind_2, inter)

    n_blocks = 10
    out = pl.pallas_call(
        _add_body,
        out_shape=jax.ShapeDtypeStruct((n_rows, d), jnp.float32),
        grid=(n_blocks,),
        in_specs=[pl.BlockSpec((_NC, n_rows // n_blocks, d),
                               lambda i: (0, i, 0))],
        out_specs=pl.BlockSpec((n_rows // n_blocks, d), lambda i: (i, 0)),
    )(part)
    return out
